# baseline (speedup 1.0000x reference)
import jax
import jax.numpy as jnp
from jax.experimental import pallas as pl
from jax.experimental.pallas import tpu as pltpu

_B, _S, _D_IN, _D_OUT = 8, 2048, 1024, 1024
_BM = 1024
_BN = 256
_NC = _D_OUT // _BN


def _mm_mask_kernel(x_ref, w_ref, b_ref, m_ref, o_ref, xm_ref):
    c = pl.program_id(1)

    @pl.when(c == 0)
    def _():
        xm_ref[...] = (x_ref[...] * m_ref[...]).astype(jnp.bfloat16)

    wblk = w_ref[:, pl.ds(c * _BN, _BN)].astype(jnp.bfloat16)
    y = jnp.dot(xm_ref[...], wblk, preferred_element_type=jnp.float32)
    o_ref[...] = y + m_ref[...] * b_ref[...]


def kernel(x, mask, W, b):
    M = _B * _S
    x2 = x.reshape(M, _D_IN)
    mf = mask.reshape(M, 1).astype(jnp.float32)
    out = pl.pallas_call(
        _mm_mask_kernel,
        grid=(M // _BM, _NC),
        in_specs=[
            pl.BlockSpec((_BM, _D_IN), lambda i, c: (i, 0)),
            pl.BlockSpec((_D_IN, _D_OUT), lambda i, c: (0, 0)),
            pl.BlockSpec((1, _BN), lambda i, c: (0, c)),
            pl.BlockSpec((_BM, 1), lambda i, c: (i, 0)),
        ],
        out_specs=pl.BlockSpec((_BM, _BN), lambda i, c: (i, c)),
        out_shape=jax.ShapeDtypeStruct((M, _D_OUT), jnp.float32),
        scratch_shapes=[pltpu.VMEM((_BM, _D_IN), jnp.bfloat16)],
        compiler_params=pltpu.CompilerParams(
            dimension_semantics=("arbitrary", "arbitrary"),
            vmem_limit_bytes=100 * 1024 * 1024,
        ),
    )(x2, W, b.reshape(1, _D_OUT), mf)
    return out.reshape(_B, _S, _D_OUT)


# speedup vs baseline: 1.6634x; 1.6634x over previous
import jax
import jax.numpy as jnp
from jax.experimental import pallas as pl
from jax.experimental.pallas import tpu as pltpu

_B, _S, _D_IN, _D_OUT = 8, 2048, 1024, 1024
_BM = 1024


def _mm_mask_kernel(x_ref, w_hbm, b_ref, m_ref, o_ref, w_f32, w_bf, sem):
    i = pl.program_id(0)

    @pl.when(i == 0)
    def _():
        cp = pltpu.make_async_copy(w_hbm, w_f32, sem)
        cp.start()
        cp.wait()
        w_bf[...] = w_f32[...].astype(jnp.bfloat16)

    m = m_ref[...]
    xm = (x_ref[...] * m).astype(jnp.bfloat16)
    y = jnp.dot(xm, w_bf[...], preferred_element_type=jnp.float32)
    o_ref[...] = y + m * b_ref[...]


def kernel(x, mask, W, b):
    M = _B * _S
    x2 = x.reshape(M, _D_IN)
    mf = mask.reshape(M, 1).astype(jnp.float32)
    out = pl.pallas_call(
        _mm_mask_kernel,
        grid=(M // _BM,),
        in_specs=[
            pl.BlockSpec((_BM, _D_IN), lambda i: (i, 0)),
            pl.BlockSpec(memory_space=pl.ANY),
            pl.BlockSpec((1, _D_OUT), lambda i: (0, 0)),
            pl.BlockSpec((_BM, 1), lambda i: (i, 0)),
        ],
        out_specs=pl.BlockSpec((_BM, _D_OUT), lambda i: (i, 0)),
        out_shape=jax.ShapeDtypeStruct((M, _D_OUT), jnp.float32),
        scratch_shapes=[
            pltpu.VMEM((_D_IN, _D_OUT), jnp.float32),
            pltpu.VMEM((_D_IN, _D_OUT), jnp.bfloat16),
            pltpu.SemaphoreType.DMA,
        ],
        compiler_params=pltpu.CompilerParams(
            dimension_semantics=("arbitrary",),
            vmem_limit_bytes=100 * 1024 * 1024,
        ),
    )(x2, W, b.reshape(1, _D_OUT), mf)
    return out.reshape(_B, _S, _D_OUT)


# speedup vs baseline: 1.7227x; 1.0356x over previous
import jax
import jax.numpy as jnp
from jax.experimental import pallas as pl
from jax.experimental.pallas import tpu as pltpu

_B, _S, _D_IN, _D_OUT = 8, 2048, 1024, 1024
_BM = 1024
_NSTEPS = (_B * _S) // _BM


def _mm_mask_kernel(x_ref, w_ref, b_ref, m_ref, o_hbm, ybuf, sems):
    i = pl.program_id(0)
    slot = jax.lax.rem(i, 2)

    def out_copy(step, s):
        dst = o_hbm.at[pl.ds(step * _BM, _BM), :]
        return pltpu.make_async_copy(ybuf.at[s], dst, sems.at[s])

    @pl.when(i >= 2)
    def _():
        out_copy(i - 2, slot).wait()

    m = m_ref[...]
    xm = (x_ref[...] * m).astype(jnp.bfloat16)
    y = jnp.dot(xm, w_ref[...].astype(jnp.bfloat16),
                preferred_element_type=jnp.float32)
    ybuf[slot] = y + m * b_ref[...]
    out_copy(i, slot).start()

    @pl.when(i == _NSTEPS - 1)
    def _():
        out_copy(i - 1, 1 - slot).wait()
        out_copy(i, slot).wait()


def kernel(x, mask, W, b):
    M = _B * _S
    x2 = x.reshape(M, _D_IN)
    mf = mask.reshape(M, 1).astype(jnp.float32)
    out = pl.pallas_call(
        _mm_mask_kernel,
        grid=(_NSTEPS,),
        in_specs=[
            pl.BlockSpec((_BM, _D_IN), lambda i: (i, 0)),
            pl.BlockSpec((_D_IN, _D_OUT), lambda i: (0, 0)),
            pl.BlockSpec((1, _D_OUT), lambda i: (0, 0)),
            pl.BlockSpec((_BM, 1), lambda i: (i, 0)),
        ],
        out_specs=pl.BlockSpec(memory_space=pl.ANY),
        out_shape=jax.ShapeDtypeStruct((M, _D_OUT), jnp.float32),
        scratch_shapes=[
            pltpu.VMEM((2, _BM, _D_OUT), jnp.float32),
            pltpu.SemaphoreType.DMA((2,)),
        ],
        compiler_params=pltpu.CompilerParams(
            dimension_semantics=("arbitrary",),
            vmem_limit_bytes=100 * 1024 * 1024,
        ),
    )(x2, W, b.reshape(1, _D_OUT), mf)
    return out.reshape(_B, _S, _D_OUT)


# speedup vs baseline: 1.8180x; 1.0553x over previous
import jax
import jax.numpy as jnp
from jax.experimental import pallas as pl

_B, _S, _D_IN, _D_OUT = 8, 2048, 1024, 1024
_BM = 2048


def _mm_mask_kernel(x_ref, w_ref, b_ref, m_ref, o_ref):
    y = jnp.dot(x_ref[...].astype(jnp.bfloat16), w_ref[...].astype(jnp.bfloat16),
                preferred_element_type=jnp.float32)
    o_ref[...] = (y + b_ref[...]) * m_ref[...]


def kernel(x, mask, W, b):
    M = _B * _S
    x2 = x.reshape(M, _D_IN)
    mf = mask.reshape(M, 1).astype(jnp.float32)
    out = pl.pallas_call(
        _mm_mask_kernel,
        grid=(M // _BM,),
        in_specs=[
            pl.BlockSpec((_BM, _D_IN), lambda i: (i, 0)),
            pl.BlockSpec((_D_IN, _D_OUT), lambda i: (0, 0)),
            pl.BlockSpec((1, _D_OUT), lambda i: (0, 0)),
            pl.BlockSpec((_BM, 1), lambda i: (i, 0)),
        ],
        out_specs=pl.BlockSpec((_BM, _D_OUT), lambda i: (i, 0)),
        out_shape=jax.ShapeDtypeStruct((M, _D_OUT), jnp.float32),
    )(x2, W, b.reshape(1, _D_OUT), mf)
    return out.reshape(_B, _S, _D_OUT)
